# baseline (speedup 1.0000x reference)
import jax
import jax.numpy as jnp
from jax import lax
from jax.experimental import pallas as pl
from jax.experimental.pallas import tpu as pltpu
from jax.experimental.pallas import tpu_sc as plsc

TAU = 0.6
MARGIN = 1.0

B = 2
D = 128
H = 256
W = 384
HW = H * W
NC = 2
NS = 16
NW = NC * NS
L = 16

P = 128
SPAN = HW // NS
HROWS = H // NS
CHUNKS = SPAN // P
TRIPLES = CHUNKS // 3
G = P // L
GI = 8
UNROLL = 4


def _anchor_terms(mv, av, dv, rv):
    wgt = jnp.clip((mv - TAU) * (1.0 / (1.0 - TAU)), 0.0, 1.0) * rv
    diff = jnp.abs(dv - av)
    viol = jnp.maximum(diff - MARGIN, 0.0)
    hub = jnp.where(viol < 1.0, 0.5 * viol * viol, viol - 0.5)
    return wgt, wgt * hub


def _sc_body(raw_hbm, mask_hbm, disp_hbm, roi_hbm, out_hbm,
             raw_v, mask_v, disp_v, roi_v, num_v, den_v,
             sr0, sr1, sr2, sm0, sm1, sm2):
    cid = lax.axis_index("c")
    sid = lax.axis_index("s")
    wid = cid * NS + sid
    row0 = cid * D
    row_h0 = sid * HROWS

    pltpu.sync_copy(disp_hbm.at[cid, pl.ds(row_h0, HROWS)], disp_v)
    pltpu.sync_copy(roi_hbm.at[cid, pl.ds(row_h0, HROWS)], roi_v)

    num_v[...] = jnp.zeros((L,), jnp.float32)
    den_v[...] = jnp.zeros((L,), jnp.float32)

    sem_r = (sr0, sr1, sr2)
    sem_m = (sm0, sm1, sm2)

    def start(slot, c):
        hr = row_h0 + c // 3
        wc = (c % 3) * 128
        idx = (pl.ds(row0, D), pl.ds(hr, 1), pl.ds(wc, P))
        pltpu.async_copy(raw_hbm.at[idx], raw_v.at[slot], sem_r[slot])
        pltpu.async_copy(mask_hbm.at[idx], mask_v.at[slot], sem_m[slot])

    def wait(slot):
        dummy = (pl.ds(0, D), pl.ds(0, 1), pl.ds(0, P))
        pltpu.make_async_copy(raw_hbm.at[dummy], raw_v.at[slot],
                              sem_r[slot]).wait()
        pltpu.make_async_copy(mask_hbm.at[dummy], mask_v.at[slot],
                              sem_m[slot]).wait()

    def compute(slot, c):
        for g0 in range(0, G, GI):
            sls = [pl.ds((g0 + gg) * L, L) for gg in range(GI)]

            def d_body(i, carry):
                ms = list(carry)
                d0 = i * UNROLL
                kc0 = jnp.uint32(0x8000007F) - d0.astype(jnp.uint32)
                for j in range(UNROLL):
                    d = d0 + j
                    kc = kc0 - jnp.uint32(j)
                    for gg in range(GI):
                        s = raw_v[slot, d, 0, sls[gg]] + \
                            (1.0 - mask_v[slot, d, 0, sls[gg]]) * (-10000.0)
                        bits = lax.bitcast_convert_type(s, jnp.uint32)
                        kk = (bits & jnp.uint32(0xFFFFFF80)) ^ kc
                        m1, m2 = ms[2 * gg], ms[2 * gg + 1]
                        t = jnp.minimum(m1, kk)
                        ms[2 * gg] = jnp.maximum(m1, kk)
                        ms[2 * gg + 1] = jnp.maximum(m2, t)
                return tuple(ms)

            init = tuple(jnp.zeros((L,), jnp.uint32) for _ in range(2 * GI))
            ms = lax.fori_loop(0, D // UNROLL, d_body, init)

            def decode(mk):
                vbits = (mk & jnp.uint32(0xFFFFFF80)) ^ jnp.uint32(0x80000000)
                val = lax.bitcast_convert_type(vbits, jnp.float32)
                ibits = (mk & jnp.uint32(0x7F)) | jnp.uint32(0x4B000000)
                idx = 8388735.0 - lax.bitcast_convert_type(ibits, jnp.float32)
                return val, idx

            for gg in range(GI):
                v1, a1 = decode(ms[2 * gg])
                v2, a2 = decode(ms[2 * gg + 1])

                px = pl.ds((c % 3) * 128 + (g0 + gg) * L, L)
                dv = disp_v[c // 3, px]
                rv = roi_v[c // 3, px]
                w1, t1 = _anchor_terms(v1, a1, dv, rv)
                w2, t2 = _anchor_terms(v2, a2, dv, rv)
                num_v[...] = num_v[...] + (t1 + t2)
                den_v[...] = den_v[...] + (w1 + w2)

    start(0, 0)
    start(1, 1)

    def ring_body(p, _):
        c0 = p * 3
        start(2, c0 + 2)
        wait(0)
        compute(0, c0)

        @pl.when(p < TRIPLES - 1)
        def _start_s0():
            start(0, c0 + 3)

        wait(1)
        compute(1, c0 + 1)

        @pl.when(p < TRIPLES - 1)
        def _start_s1():
            start(1, c0 + 4)

        wait(2)
        compute(2, c0 + 2)
        return _

    lax.fori_loop(0, TRIPLES, ring_body, None)

    pltpu.sync_copy(num_v, out_hbm.at[wid])
    pltpu.sync_copy(den_v, out_hbm.at[NW + wid])


def kernel(raw_vol, disp, mask, roi):
    raw2 = raw_vol.reshape(B * D, H, W)
    mask2 = mask.reshape(B * D, H, W)
    disp2 = disp.reshape(B, H, W)
    roi2 = roi.reshape(B, H, W)

    mesh = plsc.VectorSubcoreMesh(core_axis_name="c", subcore_axis_name="s")
    sc = pl.kernel(
        _sc_body,
        mesh=mesh,
        out_type=jax.ShapeDtypeStruct((2 * NW, L), jnp.float32),
        scratch_types=[
            pltpu.VMEM((3, D, 1, P), jnp.float32),
            pltpu.VMEM((3, D, 1, P), jnp.float32),
            pltpu.VMEM((HROWS, W), jnp.float32),
            pltpu.VMEM((HROWS, W), jnp.float32),
            pltpu.VMEM((L,), jnp.float32),
            pltpu.VMEM((L,), jnp.float32),
            pltpu.SemaphoreType.DMA,
            pltpu.SemaphoreType.DMA,
            pltpu.SemaphoreType.DMA,
            pltpu.SemaphoreType.DMA,
            pltpu.SemaphoreType.DMA,
            pltpu.SemaphoreType.DMA,
        ],
    )
    part = sc(raw2, mask2, disp2, roi2)
    num = part[:NW].sum()
    den = part[NW:].sum()
    return num / (den + 1e-6)


# speedup vs baseline: 1.0270x; 1.0270x over previous
import jax
import jax.numpy as jnp
from jax import lax
from jax.experimental import pallas as pl
from jax.experimental.pallas import tpu as pltpu
from jax.experimental.pallas import tpu_sc as plsc

TAU = 0.6
MARGIN = 1.0

B = 2
D = 128
H = 256
W = 384
HW = H * W
NC = 2
NS = 16
NW = NC * NS
L = 16

P = 128
SPAN = HW // NS
HROWS = H // NS
CHUNKS = SPAN // P
TRIPLES = CHUNKS // 3
G = P // L
GI = 8
UNROLL = 2


def _anchor_terms(mv, av, dv, rv):
    wgt = jnp.clip((mv - TAU) * (1.0 / (1.0 - TAU)), 0.0, 1.0) * rv
    diff = jnp.abs(dv - av)
    viol = jnp.maximum(diff - MARGIN, 0.0)
    hub = jnp.where(viol < 1.0, 0.5 * viol * viol, viol - 0.5)
    return wgt, wgt * hub


def _sc_body(raw_hbm, mask_hbm, disp_hbm, roi_hbm, out_hbm,
             raw_v, mask_v, disp_v, roi_v, num_v, den_v,
             sr0, sr1, sr2, sm0, sm1, sm2):
    cid = lax.axis_index("c")
    sid = lax.axis_index("s")
    wid = cid * NS + sid
    row0 = cid * D
    row_h0 = sid * HROWS

    pltpu.sync_copy(disp_hbm.at[cid, pl.ds(row_h0, HROWS)], disp_v)
    pltpu.sync_copy(roi_hbm.at[cid, pl.ds(row_h0, HROWS)], roi_v)

    num_v[...] = jnp.zeros((L,), jnp.float32)
    den_v[...] = jnp.zeros((L,), jnp.float32)

    sem_r = (sr0, sr1, sr2)
    sem_m = (sm0, sm1, sm2)

    def start(slot, c):
        hr = row_h0 + c // 3
        wc = (c % 3) * 128
        idx = (pl.ds(row0, D), pl.ds(hr, 1), pl.ds(wc, P))
        pltpu.async_copy(raw_hbm.at[idx], raw_v.at[slot], sem_r[slot])
        pltpu.async_copy(mask_hbm.at[idx], mask_v.at[slot], sem_m[slot])

    def wait(slot):
        dummy = (pl.ds(0, D), pl.ds(0, 1), pl.ds(0, P))
        pltpu.make_async_copy(raw_hbm.at[dummy], raw_v.at[slot],
                              sem_r[slot]).wait()
        pltpu.make_async_copy(mask_hbm.at[dummy], mask_v.at[slot],
                              sem_m[slot]).wait()

    def compute(slot, c):
        for g0 in range(0, G, GI):
            sls = [pl.ds((g0 + gg) * L, L) for gg in range(GI)]

            def d_body(i, carry):
                ms = list(carry)
                d0 = i * UNROLL
                kc0 = jnp.uint32(0x8000007F) - d0.astype(jnp.uint32)
                for j in range(UNROLL):
                    d = d0 + j
                    kc = kc0 - jnp.uint32(j)
                    for gg in range(GI):
                        s = raw_v[slot, d, 0, sls[gg]] + \
                            (1.0 - mask_v[slot, d, 0, sls[gg]]) * (-10000.0)
                        bits = lax.bitcast_convert_type(s, jnp.uint32)
                        kk = (bits & jnp.uint32(0xFFFFFF80)) ^ kc
                        m1, m2 = ms[2 * gg], ms[2 * gg + 1]
                        t = jnp.minimum(m1, kk)
                        ms[2 * gg] = jnp.maximum(m1, kk)
                        ms[2 * gg + 1] = jnp.maximum(m2, t)
                return tuple(ms)

            init = tuple(jnp.zeros((L,), jnp.uint32) for _ in range(2 * GI))
            ms = lax.fori_loop(0, D // UNROLL, d_body, init)

            def decode(mk):
                vbits = (mk & jnp.uint32(0xFFFFFF80)) ^ jnp.uint32(0x80000000)
                val = lax.bitcast_convert_type(vbits, jnp.float32)
                ibits = (mk & jnp.uint32(0x7F)) | jnp.uint32(0x4B000000)
                idx = 8388735.0 - lax.bitcast_convert_type(ibits, jnp.float32)
                return val, idx

            for gg in range(GI):
                v1, a1 = decode(ms[2 * gg])
                v2, a2 = decode(ms[2 * gg + 1])

                px = pl.ds((c % 3) * 128 + (g0 + gg) * L, L)
                dv = disp_v[c // 3, px]
                rv = roi_v[c // 3, px]
                w1, t1 = _anchor_terms(v1, a1, dv, rv)
                w2, t2 = _anchor_terms(v2, a2, dv, rv)
                num_v[...] = num_v[...] + (t1 + t2)
                den_v[...] = den_v[...] + (w1 + w2)

    start(0, 0)
    start(1, 1)

    def ring_body(p, _):
        c0 = p * 3
        start(2, c0 + 2)
        wait(0)
        compute(0, c0)

        @pl.when(p < TRIPLES - 1)
        def _start_s0():
            start(0, c0 + 3)

        wait(1)
        compute(1, c0 + 1)

        @pl.when(p < TRIPLES - 1)
        def _start_s1():
            start(1, c0 + 4)

        wait(2)
        compute(2, c0 + 2)
        return _

    lax.fori_loop(0, TRIPLES, ring_body, None)

    pltpu.sync_copy(num_v, out_hbm.at[wid])
    pltpu.sync_copy(den_v, out_hbm.at[NW + wid])


def kernel(raw_vol, disp, mask, roi):
    raw2 = raw_vol.reshape(B * D, H, W)
    mask2 = mask.reshape(B * D, H, W)
    disp2 = disp.reshape(B, H, W)
    roi2 = roi.reshape(B, H, W)

    mesh = plsc.VectorSubcoreMesh(core_axis_name="c", subcore_axis_name="s")
    sc = pl.kernel(
        _sc_body,
        mesh=mesh,
        out_type=jax.ShapeDtypeStruct((2 * NW, L), jnp.float32),
        scratch_types=[
            pltpu.VMEM((3, D, 1, P), jnp.float32),
            pltpu.VMEM((3, D, 1, P), jnp.float32),
            pltpu.VMEM((HROWS, W), jnp.float32),
            pltpu.VMEM((HROWS, W), jnp.float32),
            pltpu.VMEM((L,), jnp.float32),
            pltpu.VMEM((L,), jnp.float32),
            pltpu.SemaphoreType.DMA,
            pltpu.SemaphoreType.DMA,
            pltpu.SemaphoreType.DMA,
            pltpu.SemaphoreType.DMA,
            pltpu.SemaphoreType.DMA,
            pltpu.SemaphoreType.DMA,
        ],
    )
    part = sc(raw2, mask2, disp2, roi2)
    num = part[:NW].sum()
    den = part[NW:].sum()
    return num / (den + 1e-6)
